# baseline (speedup 1.0000x reference)
import functools

import jax
import jax.numpy as jnp
from jax import lax
from jax.experimental import pallas as pl
from jax.experimental.pallas import tpu as pltpu
from jax.experimental.pallas import tpu_sc as plsc

NC = 2
NS = 16
NW = NC * NS
B = 16384
D = 64
BPW = B // NW
CHUNK = 128
NCHUNK = BPW // CHUNK


def _gather_body(users_hbm, isbns_hbm, utab_hbm, itab_hbm,
                 uout_hbm, iout_hbm,
                 uidx_v, iidx_v, urows_v, irows_v, usem, isem):
    wid = lax.axis_index("s") * NC + lax.axis_index("c")
    base = wid * BPW
    for j in range(NCHUNK):
        pltpu.sync_copy(users_hbm.at[pl.ds(base + j * CHUNK, CHUNK)],
                        uidx_v.at[j])
        pltpu.sync_copy(isbns_hbm.at[pl.ds(base + j * CHUNK, CHUNK)],
                        iidx_v.at[j])
    ucopies = [
        pltpu.async_copy(utab_hbm.at[uidx_v.at[j]],
                         urows_v.at[pl.ds(j * CHUNK, CHUNK)], usem)
        for j in range(NCHUNK)
    ]
    icopies = [
        pltpu.async_copy(itab_hbm.at[iidx_v.at[j]],
                         irows_v.at[pl.ds(j * CHUNK, CHUNK)], isem)
        for j in range(NCHUNK)
    ]
    for c in ucopies:
        c.wait()
    pltpu.sync_copy(urows_v, uout_hbm.at[pl.ds(base, BPW)])
    for c in icopies:
        c.wait()
    pltpu.sync_copy(irows_v, iout_hbm.at[pl.ds(base, BPW)])


def _sc_gather(users, isbns, user_table, isbn_table):
    mesh = plsc.VectorSubcoreMesh(core_axis_name="c", subcore_axis_name="s")
    f = pl.kernel(
        _gather_body,
        out_type=(
            jax.ShapeDtypeStruct((B, D), jnp.float32),
            jax.ShapeDtypeStruct((B, D), jnp.float32),
        ),
        mesh=mesh,
        compiler_params=pltpu.CompilerParams(use_tc_tiling_on_sc=False),
        scratch_types=[
            pltpu.VMEM((NCHUNK, CHUNK), jnp.int32),
            pltpu.VMEM((NCHUNK, CHUNK), jnp.int32),
            pltpu.VMEM((BPW, D), jnp.float32),
            pltpu.VMEM((BPW, D), jnp.float32),
            pltpu.SemaphoreType.DMA,
            pltpu.SemaphoreType.DMA,
        ],
    )
    return f(users, isbns, user_table, isbn_table)


BM = 1024


def _mlp_body(u_ref, i_ref, w1u_ref, w1i_ref, b1_ref, w2_ref, b2_ref,
              w3_ref, b3_ref, o_ref):
    x = jnp.dot(u_ref[...], w1u_ref[...], preferred_element_type=jnp.float32)
    x = x + jnp.dot(i_ref[...], w1i_ref[...],
                    preferred_element_type=jnp.float32)
    x = jnp.maximum(x + b1_ref[...], 0.0)
    x = jnp.maximum(
        jnp.dot(x, w2_ref[...], preferred_element_type=jnp.float32)
        + b2_ref[...], 0.0)
    o_ref[...] = (jnp.dot(x, w3_ref[...], preferred_element_type=jnp.float32)
                  + b3_ref[...])


def _tc_mlp(u_emb, i_emb, W1, b1, W2, b2, W3, b3):
    w1u = W1[:D]
    w1i = W1[D:]
    full = lambda s: pl.BlockSpec(s, lambda m: (0, 0))
    return pl.pallas_call(
        _mlp_body,
        grid=(B // BM,),
        in_specs=[
            pl.BlockSpec((BM, D), lambda m: (m, 0)),
            pl.BlockSpec((BM, D), lambda m: (m, 0)),
            full((D, 64)),
            full((D, 64)),
            full((1, 64)),
            full((64, 32)),
            full((1, 32)),
            full((32, 1)),
            full((1, 1)),
        ],
        out_specs=pl.BlockSpec((BM, 1), lambda m: (m, 0)),
        out_shape=jax.ShapeDtypeStruct((B, 1), jnp.float32),
    )(u_emb, i_emb, w1u, w1i, b1.reshape(1, 64), W2, b2.reshape(1, 32),
      W3, b3.reshape(1, 1))


def kernel(users, isbns, user_table, isbn_table, W1, b1, W2, b2, W3, b3):
    u_emb, i_emb = _sc_gather(users, isbns, user_table, isbn_table)
    return _tc_mlp(u_emb, i_emb, W1, b1, W2, b2, W3, b3)
